# baseline (speedup 1.0000x reference)
import functools

import jax
import jax.numpy as jnp
from jax import lax
from jax.experimental import pallas as pl
from jax.experimental.pallas import tpu as pltpu
from jax.experimental.pallas import tpu_sc as plsc

FROM_DIM = 64
TO_DIM = 64
BATCH = 16384
FIELDS = 26

B_FLAT = BATCH * FIELDS
NC, NS = 2, 16
NW = NC * NS
RW = B_FLAT // NW
CH = 128
NCH = RW // CH

MM_BLK = 8192


def _sc_gather(table, idx_flat):
    mesh = plsc.VectorSubcoreMesh(
        core_axis_name="c", subcore_axis_name="s",
        num_cores=NC, num_subcores=NS)

    @functools.partial(
        pl.kernel,
        out_type=jax.ShapeDtypeStruct((B_FLAT, FROM_DIM), jnp.float32),
        mesh=mesh,
        scratch_types=[
            pltpu.VMEM((RW,), jnp.int32),
            pltpu.VMEM((CH, FROM_DIM), jnp.float32),
            pltpu.SemaphoreType.DMA,
        ],
    )
    def gather_kernel(table_hbm, idx_hbm, emb_hbm, idx_v, rows_v, sem):
        wid = lax.axis_index("s") * NC + lax.axis_index("c")
        r0 = wid * RW
        pltpu.sync_copy(idx_hbm.at[pl.ds(r0, RW)], idx_v)

        def chunk(g, carry):
            base = g * CH
            descs = []
            for jb in range(CH // 16):
                vi = idx_v[pl.ds(base + jb * 16, 16)]
                for j in range(16):
                    descs.append(pltpu.async_copy(
                        table_hbm.at[pl.ds(vi[j], 1)],
                        rows_v.at[pl.ds(jb * 16 + j, 1)], sem))
            for d in descs:
                d.wait()
            pltpu.sync_copy(rows_v, emb_hbm.at[pl.ds(r0 + base, CH)])
            return carry

        lax.fori_loop(0, NCH, chunk, 0)

    return gather_kernel(table, idx_flat)


def _mm_body(x_ref, wt_ref, b_ref, o_ref):
    o_ref[...] = (
        jnp.dot(x_ref[...], wt_ref[...], preferred_element_type=jnp.float32)
        + b_ref[...]
    )


def _tc_linear(emb, Wt, b):
    return pl.pallas_call(
        _mm_body,
        grid=(B_FLAT // MM_BLK,),
        in_specs=[
            pl.BlockSpec((MM_BLK, FROM_DIM), lambda i: (i, 0)),
            pl.BlockSpec((FROM_DIM, TO_DIM), lambda i: (0, 0)),
            pl.BlockSpec((1, TO_DIM), lambda i: (0, 0)),
        ],
        out_specs=pl.BlockSpec((MM_BLK, TO_DIM), lambda i: (i, 0)),
        out_shape=jax.ShapeDtypeStruct((B_FLAT, TO_DIM), jnp.float32),
    )(emb, Wt, b)


def kernel(indexes, table, W, b):
    idx_flat = indexes.astype(jnp.int32).reshape(B_FLAT)
    emb = _sc_gather(table, idx_flat)
    out = _tc_linear(emb, W.T, b.reshape(1, TO_DIM))
    return out.reshape(BATCH, FIELDS, TO_DIM)


# speedup vs baseline: 1.1129x; 1.1129x over previous
import functools

import jax
import jax.numpy as jnp
from jax import lax
from jax.experimental import pallas as pl
from jax.experimental.pallas import tpu as pltpu
from jax.experimental.pallas import tpu_sc as plsc

FROM_DIM = 64
TO_DIM = 64
BATCH = 16384
FIELDS = 26

NC, NS = 2, 16
NW = NC * NS
BW = BATCH // NW
BB = 16
NCH = BW // BB

MM_BM = 256


def _sc_gather(table, idx2d):
    mesh = plsc.VectorSubcoreMesh(
        core_axis_name="c", subcore_axis_name="s",
        num_cores=NC, num_subcores=NS)

    @functools.partial(
        pl.kernel,
        out_type=jax.ShapeDtypeStruct((BATCH, FIELDS, FROM_DIM), jnp.float32),
        mesh=mesh,
        scratch_types=[
            pltpu.VMEM((BB, FIELDS), jnp.int32),
            pltpu.VMEM((BB, FIELDS, FROM_DIM), jnp.float32),
            pltpu.SemaphoreType.DMA,
            pltpu.SemaphoreType.DMA,
        ],
    )
    def gather_kernel(table_hbm, idx_hbm, emb_hbm, idx_v, rows_v, isem, sem):
        wid = lax.axis_index("s") * NC + lax.axis_index("c")
        b0 = wid * BW

        def chunk(g, carry):
            bb0 = b0 + g * BB
            pltpu.sync_copy(idx_hbm.at[pl.ds(bb0, BB)], idx_v)
            for bb in range(BB):
                v0 = idx_v[bb, pl.ds(0, 16)]
                v1 = idx_v[bb, pl.ds(FIELDS - 16, 16)]
                for f in range(FIELDS):
                    s = v0[f] if f < 16 else v1[f - (FIELDS - 16)]
                    pltpu.async_copy(
                        table_hbm.at[pl.ds(s, 1)],
                        rows_v.at[bb].at[pl.ds(f, 1)], sem)
            pltpu.make_async_copy(
                emb_hbm.at[pl.ds(0, BB)], rows_v, sem).wait()
            pltpu.sync_copy(rows_v, emb_hbm.at[pl.ds(bb0, BB)])
            return carry

        lax.fori_loop(0, NCH, chunk, 0)

    return gather_kernel(table, idx2d)


def _mm_body(x_ref, wt_ref, b_ref, o_ref):
    x = x_ref[...]
    y = lax.dot_general(
        x, wt_ref[...],
        dimension_numbers=(((2,), (0,)), ((), ())),
        preferred_element_type=jnp.float32,
    )
    o_ref[...] = y + b_ref[...]


def _tc_linear(emb, Wt, b):
    return pl.pallas_call(
        _mm_body,
        grid=(BATCH // MM_BM,),
        in_specs=[
            pl.BlockSpec((MM_BM, FIELDS, FROM_DIM), lambda i: (i, 0, 0)),
            pl.BlockSpec((FROM_DIM, TO_DIM), lambda i: (0, 0)),
            pl.BlockSpec((1, 1, TO_DIM), lambda i: (0, 0, 0)),
        ],
        out_specs=pl.BlockSpec((MM_BM, FIELDS, TO_DIM), lambda i: (i, 0, 0)),
        out_shape=jax.ShapeDtypeStruct((BATCH, FIELDS, TO_DIM), jnp.float32),
    )(emb, Wt, b)


def kernel(indexes, table, W, b):
    emb = _sc_gather(table, indexes.astype(jnp.int32))
    return _tc_linear(emb, W.T, b.reshape(1, 1, TO_DIM))
